# baseline (speedup 1.0000x reference)
import functools

import jax
import jax.numpy as jnp
from jax import lax
from jax.experimental import pallas as pl
from jax.experimental.pallas import tpu as pltpu
from jax.experimental.pallas import tpu_sc as plsc

N_NODES = 10000
N_EDGES = 320000
DIM = 128
N_RELS = 8

NC = 2
NS = 16
NW = NC * NS
GB = 128
G = 80
CH = 16
NCH = G // CH
EPT = G * GB
E_PAD = NW * EPT
ACC_N = 10240
ROWS_PT = ACC_N // NS

_SC_MESH = plsc.VectorSubcoreMesh(
    core_axis_name="c", subcore_axis_name="s", num_cores=NC, num_subcores=NS)


def _sc_body(hrel, gidx, dste, nrm, out, gidx_v, dst_v, norm_v, rows_v, acc,
             sg0, sg1, ss0, ss1, st0, st1):
  c = lax.axis_index("c")
  s = lax.axis_index("s")
  w = s * NC + c
  sg = (sg0, sg1)
  ss = (ss0, ss1)
  st = (st0, st1)

  def _refill(ch, buf):
    sl = pl.ds(ch * CH, CH)
    pltpu.async_copy(gidx.at[w, sl], gidx_v.at[buf], st[buf])
    pltpu.async_copy(dste.at[w, sl], dst_v.at[buf], st[buf])
    pltpu.async_copy(nrm.at[w, sl], norm_v.at[buf], st[buf])

  def _wait_refill(buf):
    for _ in range(2):
      pltpu.make_async_copy(gidx.at[w, pl.ds(0, CH)], gidx_v.at[buf],
                            st[buf]).wait()
    pltpu.make_async_copy(nrm.at[w, pl.ds(0, CH)], norm_v.at[buf],
                          st[buf]).wait()

  _refill(0, 0)
  _refill(1, 1)
  zero = jnp.zeros((16,), jnp.float32)

  def _zero_rows(e, carry):
    for j in range(DIM // 16):
      rows_v[0, e, pl.ds(j * 16, 16)] = zero
    return carry

  lax.fori_loop(0, GB, _zero_rows, 0)
  base = s * ROWS_PT
  for k in range(ROWS_PT // GB):
    pltpu.async_copy(rows_v.at[0], acc.at[pl.ds(base + k * GB, GB)], ss0)
  for k in range(ROWS_PT // GB):
    pltpu.make_async_copy(rows_v.at[0], acc.at[pl.ds(base, GB)], ss0).wait()
  _wait_refill(0)

  pltpu.async_copy(hrel.at[gidx_v.at[0, 0]], rows_v.at[0], sg0)
  plsc.subcore_barrier()

  def _scale(b, buf, gm):
    def _scale16(e16, carry2):
      nv = norm_v[buf, gm, pl.ds(e16 * 16, 16)]
      for l in range(16):
        nb = jnp.full((16,), nv[l], jnp.float32)
        e = e16 * 16 + l
        for j in range(DIM // 16):
          sl = pl.ds(j * 16, 16)
          rows_v[b, e, sl] = rows_v[b, e, sl] * nb
      return carry2

    lax.fori_loop(0, GB // 16, _scale16, 0)

  def _step(g, gm, k, buf, nbuf, gm1, last):
    bn = 1 - k
    if not (isinstance(g, int) and g == 0):
      pltpu.make_async_copy(rows_v.at[bn], acc.at[dst_v.at[0, 0]],
                            ss[bn]).wait()
    if not last:
      pltpu.async_copy(hrel.at[gidx_v.at[nbuf, gm1]], rows_v.at[bn], sg[bn])
    pltpu.make_async_copy(hrel.at[gidx_v.at[0, 0]], rows_v.at[k], sg[k]).wait()
    _scale(k, buf, gm)
    pltpu.async_copy(rows_v.at[k], acc.at[dst_v.at[buf, gm]], ss[k], add=True)

  for ch in range(NCH):
    buf = ch % 2
    g0 = ch * CH
    _step(g0, 0, 0, buf, buf, 1, False)
    _step(g0 + 1, 1, 1, buf, buf, 2, False)
    if 1 <= ch < NCH - 1:
      _refill(ch + 1, 1 - buf)

    def _mid(i, carry):
      gm = 2 + 2 * i
      _step(g0 + gm, gm, 0, buf, buf, gm + 1, False)
      _step(g0 + gm + 1, gm + 1, 1, buf, buf, gm + 2, False)
      return carry

    lax.fori_loop(0, (CH - 4) // 2, _mid, 0)
    if ch + 1 < NCH:
      _wait_refill(1 - buf)
    _step(g0 + CH - 2, CH - 2, 0, buf, buf, CH - 1, False)
    _step(g0 + CH - 1, CH - 1, 1, buf, 1 - buf, 0, ch + 1 == NCH)

  pltpu.make_async_copy(rows_v.at[1], acc.at[dst_v.at[0, 0]], ss[1]).wait()

  plsc.subcore_barrier()

  pltpu.sync_copy(acc.at[pl.ds(base, ROWS_PT)], out.at[c, pl.ds(base, ROWS_PT)])


_sc_gather_scatter = functools.partial(
    pl.kernel,
    out_type=jax.ShapeDtypeStruct((NC, ACC_N, DIM), jnp.float32),
    mesh=_SC_MESH,
    scratch_types=[
        pltpu.VMEM((2, CH, GB), jnp.int32),
        pltpu.VMEM((2, CH, GB), jnp.int32),
        pltpu.VMEM((2, CH, GB), jnp.float32),
        pltpu.VMEM((2, GB, DIM), jnp.float32),
        pltpu.VMEM_SHARED((ACC_N, DIM), jnp.float32),
        pltpu.SemaphoreType.DMA,
        pltpu.SemaphoreType.DMA,
        pltpu.SemaphoreType.DMA,
        pltpu.SemaphoreType.DMA,
        pltpu.SemaphoreType.DMA,
        pltpu.SemaphoreType.DMA,
    ],
)(_sc_body)


BN = 1000
NB = N_NODES // BN


def _tc_transform_body(x_ref, w_ref, b_ref, hrel_ref, sl_ref):
  r = pl.program_id(1)
  acc = jnp.dot(x_ref[...], w_ref[0], preferred_element_type=jnp.float32)

  @pl.when(r < N_RELS)
  def _():
    hrel_ref[0] = acc

  @pl.when(r == N_RELS)
  def _():
    sl_ref[...] = acc + b_ref[0]


def _tc_transform(x, wall, bias):
  return pl.pallas_call(
      _tc_transform_body,
      grid=(NB, N_RELS + 1),
      in_specs=[
          pl.BlockSpec((BN, DIM), lambda i, r: (i, 0)),
          pl.BlockSpec((1, DIM, DIM), lambda i, r: (r, 0, 0)),
          pl.BlockSpec((1, DIM), lambda i, r: (0, 0)),
      ],
      out_specs=[
          pl.BlockSpec((1, BN, DIM), lambda i, r: (jnp.minimum(r, N_RELS - 1), i, 0)),
          pl.BlockSpec((BN, DIM), lambda i, r: (i, 0)),
      ],
      out_shape=[
          jax.ShapeDtypeStruct((N_RELS, N_NODES, DIM), jnp.float32),
          jax.ShapeDtypeStruct((N_NODES, DIM), jnp.float32),
      ],
  )(x, wall, bias)


def _tc_fuse_transform_body(a_ref, sl_ref, w_ref, b_ref, hrel_ref, sl2_ref):
  r = pl.program_id(1)
  h = jnp.maximum(a_ref[0] + a_ref[1] + sl_ref[...], 0.0)
  acc = jnp.dot(h, w_ref[0], preferred_element_type=jnp.float32)

  @pl.when(r < N_RELS)
  def _():
    hrel_ref[0] = acc

  @pl.when(r == N_RELS)
  def _():
    sl2_ref[...] = acc + b_ref[0]


def _tc_fuse_transform(agg, sl, wall, bias):
  return pl.pallas_call(
      _tc_fuse_transform_body,
      grid=(NB, N_RELS + 1),
      in_specs=[
          pl.BlockSpec((NC, BN, DIM), lambda i, r: (0, i, 0)),
          pl.BlockSpec((BN, DIM), lambda i, r: (i, 0)),
          pl.BlockSpec((1, DIM, DIM), lambda i, r: (r, 0, 0)),
          pl.BlockSpec((1, DIM), lambda i, r: (0, 0)),
      ],
      out_specs=[
          pl.BlockSpec((1, BN, DIM), lambda i, r: (jnp.minimum(r, N_RELS - 1), i, 0)),
          pl.BlockSpec((BN, DIM), lambda i, r: (i, 0)),
      ],
      out_shape=[
          jax.ShapeDtypeStruct((N_RELS, N_NODES, DIM), jnp.float32),
          jax.ShapeDtypeStruct((N_NODES, DIM), jnp.float32),
      ],
  )(agg, sl, wall, bias)


def _tc_final_body(a_ref, sl_ref, out_ref):
  out_ref[...] = a_ref[0] + a_ref[1] + sl_ref[...]


def _tc_final(agg, sl):
  return pl.pallas_call(
      _tc_final_body,
      grid=(NB,),
      in_specs=[
          pl.BlockSpec((NC, BN, DIM), lambda i: (0, i, 0)),
          pl.BlockSpec((BN, DIM), lambda i: (i, 0)),
      ],
      out_specs=pl.BlockSpec((BN, DIM), lambda i: (i, 0)),
      out_shape=jax.ShapeDtypeStruct((N_NODES, DIM), jnp.float32),
  )(agg, sl)


def kernel(features, edge_index, etypes, norm, W1, loop1, b1, W2, loop2, b2):
  src = edge_index[0].astype(jnp.int32)
  dst = edge_index[1].astype(jnp.int32)
  et = etypes.astype(jnp.int32)
  gidx = et * N_NODES + src

  pad = E_PAD - N_EDGES
  gidx_p = jnp.concatenate([gidx, jnp.zeros((pad,), jnp.int32)]).reshape(NW, G, GB)
  dst_p = jnp.concatenate([dst, jnp.zeros((pad,), jnp.int32)]).reshape(NW, G, GB)
  norm_p = jnp.concatenate(
      [norm.reshape(N_EDGES), jnp.zeros((pad,), jnp.float32)]).reshape(NW, G, GB)

  wall1 = jnp.concatenate([W1, loop1[None]], axis=0)
  wall2 = jnp.concatenate([W2, loop2[None]], axis=0)

  hrel1, sl1 = _tc_transform(features, wall1, b1[None])
  agg1 = _sc_gather_scatter(hrel1.reshape(N_RELS * N_NODES, DIM),
                            gidx_p, dst_p, norm_p)
  hrel2, sl2 = _tc_fuse_transform(agg1, sl1, wall2, b2[None])
  agg2 = _sc_gather_scatter(hrel2.reshape(N_RELS * N_NODES, DIM),
                            gidx_p, dst_p, norm_p)
  return _tc_final(agg2, sl2)


# speedup vs baseline: 1.0588x; 1.0588x over previous
import functools

import jax
import jax.numpy as jnp
from jax import lax
from jax.experimental import pallas as pl
from jax.experimental.pallas import tpu as pltpu
from jax.experimental.pallas import tpu_sc as plsc

N_NODES = 10000
N_EDGES = 320000
DIM = 128
N_RELS = 8

NC = 2
NS = 16
NW = NC * NS
GB = 128
G = 80
CH = 16
NCH = G // CH
EPT = G * GB
E_PAD = NW * EPT
ACC_N = 10240
ROWS_PT = ACC_N // NS

_SC_MESH = plsc.VectorSubcoreMesh(
    core_axis_name="c", subcore_axis_name="s", num_cores=NC, num_subcores=NS)


def _sc_body(hrel, gidx, dste, nrm, out, gidx_v, dst_v, norm_v, rows_v, acc,
             sg0, sg1, ss0, ss1, st0, st1):
  c = lax.axis_index("c")
  s = lax.axis_index("s")
  w = s * NC + c
  sg = (sg0, sg1)
  ss = (ss0, ss1)
  st = (st0, st1)

  def _refill(ch, buf):
    sl = pl.ds(ch * CH, CH)
    pltpu.async_copy(gidx.at[w, sl], gidx_v.at[buf], st[buf])
    pltpu.async_copy(dste.at[w, sl], dst_v.at[buf], st[buf])
    pltpu.async_copy(nrm.at[w, sl], norm_v.at[buf], st[buf])

  def _wait_refill(buf):
    for _ in range(2):
      pltpu.make_async_copy(gidx.at[w, pl.ds(0, CH)], gidx_v.at[buf],
                            st[buf]).wait()
    pltpu.make_async_copy(nrm.at[w, pl.ds(0, CH)], norm_v.at[buf],
                          st[buf]).wait()

  _refill(0, 0)
  _refill(1, 1)
  zero = jnp.zeros((16,), jnp.float32)

  def _zero_rows(e, carry):
    for j in range(DIM // 16):
      rows_v[0, e, pl.ds(j * 16, 16)] = zero
    return carry

  lax.fori_loop(0, GB, _zero_rows, 0)
  base = s * ROWS_PT
  for k in range(ROWS_PT // GB):
    pltpu.async_copy(rows_v.at[0], acc.at[pl.ds(base + k * GB, GB)], ss0)
  for k in range(ROWS_PT // GB):
    pltpu.make_async_copy(rows_v.at[0], acc.at[pl.ds(base, GB)], ss0).wait()
  _wait_refill(0)

  pltpu.async_copy(hrel.at[gidx_v.at[0, 0]], rows_v.at[0], sg0)
  plsc.subcore_barrier()

  def _scale(b, buf, gm):
    def _scale16(e16, carry2):
      nv = norm_v[buf, gm, pl.ds(e16 * 16, 16)]
      for l in range(16):
        nb = jnp.full((16,), nv[l], jnp.float32)
        e = e16 * 16 + l
        for j in range(DIM // 16):
          sl = pl.ds(j * 16, 16)
          rows_v[b, e, sl] = rows_v[b, e, sl] * nb
      return carry2

    lax.fori_loop(0, GB // 16, _scale16, 0)

  def _step(g, gm, k, buf, nbuf, gm1, last):
    bn = 1 - k
    if not (isinstance(g, int) and g == 0):
      pltpu.make_async_copy(rows_v.at[bn], acc.at[dst_v.at[0, 0]],
                            ss[bn]).wait()
    if not last:
      pltpu.async_copy(hrel.at[gidx_v.at[nbuf, gm1]], rows_v.at[bn], sg[bn])
    pltpu.make_async_copy(hrel.at[gidx_v.at[0, 0]], rows_v.at[k], sg[k]).wait()
    _scale(k, buf, gm)
    pltpu.async_copy(rows_v.at[k], acc.at[dst_v.at[buf, gm]], ss[k], add=True)

  for ch in range(NCH):
    buf = ch % 2
    g0 = ch * CH
    _step(g0, 0, 0, buf, buf, 1, False)
    _step(g0 + 1, 1, 1, buf, buf, 2, False)
    if 1 <= ch < NCH - 1:
      _refill(ch + 1, 1 - buf)

    def _mid(i, carry):
      gm = 2 + 2 * i
      _step(g0 + gm, gm, 0, buf, buf, gm + 1, False)
      _step(g0 + gm + 1, gm + 1, 1, buf, buf, gm + 2, False)
      return carry

    lax.fori_loop(0, (CH - 4) // 2, _mid, 0)
    if ch + 1 < NCH:
      _wait_refill(1 - buf)
    _step(g0 + CH - 2, CH - 2, 0, buf, buf, CH - 1, False)
    _step(g0 + CH - 1, CH - 1, 1, buf, 1 - buf, 0, ch + 1 == NCH)

  pltpu.make_async_copy(rows_v.at[1], acc.at[dst_v.at[0, 0]], ss[1]).wait()

  plsc.subcore_barrier()

  pltpu.sync_copy(acc.at[pl.ds(base, ROWS_PT)], out.at[c, pl.ds(base, ROWS_PT)])


_sc_gather_scatter = functools.partial(
    pl.kernel,
    out_type=jax.ShapeDtypeStruct((NC, ACC_N, DIM), jnp.float32),
    mesh=_SC_MESH,
    scratch_types=[
        pltpu.VMEM((2, CH, GB), jnp.int32),
        pltpu.VMEM((2, CH, GB), jnp.int32),
        pltpu.VMEM((2, CH, GB), jnp.float32),
        pltpu.VMEM((2, GB, DIM), jnp.float32),
        pltpu.VMEM_SHARED((ACC_N, DIM), jnp.float32),
        pltpu.SemaphoreType.DMA,
        pltpu.SemaphoreType.DMA,
        pltpu.SemaphoreType.DMA,
        pltpu.SemaphoreType.DMA,
        pltpu.SemaphoreType.DMA,
        pltpu.SemaphoreType.DMA,
    ],
)(_sc_body)


BN = 2000
NB = N_NODES // BN


def _tc_transform_body(x_ref, w_ref, b_ref, hrel_ref, sl_ref):
  r = pl.program_id(1)
  acc = jnp.dot(x_ref[...], w_ref[0], preferred_element_type=jnp.float32)

  @pl.when(r < N_RELS)
  def _():
    hrel_ref[0] = acc

  @pl.when(r == N_RELS)
  def _():
    sl_ref[...] = acc + b_ref[0]


def _tc_transform(x, wall, bias):
  return pl.pallas_call(
      _tc_transform_body,
      grid=(NB, N_RELS + 1),
      in_specs=[
          pl.BlockSpec((BN, DIM), lambda i, r: (i, 0)),
          pl.BlockSpec((1, DIM, DIM), lambda i, r: (r, 0, 0)),
          pl.BlockSpec((1, DIM), lambda i, r: (0, 0)),
      ],
      out_specs=[
          pl.BlockSpec((1, BN, DIM), lambda i, r: (jnp.minimum(r, N_RELS - 1), i, 0)),
          pl.BlockSpec((BN, DIM), lambda i, r: (i, 0)),
      ],
      out_shape=[
          jax.ShapeDtypeStruct((N_RELS, N_NODES, DIM), jnp.float32),
          jax.ShapeDtypeStruct((N_NODES, DIM), jnp.float32),
      ],
  )(x, wall, bias)


def _tc_fuse_transform_body(a_ref, sl_ref, w_ref, b_ref, hrel_ref, sl2_ref):
  r = pl.program_id(1)
  h = jnp.maximum(a_ref[0] + a_ref[1] + sl_ref[...], 0.0)
  acc = jnp.dot(h, w_ref[0], preferred_element_type=jnp.float32)

  @pl.when(r < N_RELS)
  def _():
    hrel_ref[0] = acc

  @pl.when(r == N_RELS)
  def _():
    sl2_ref[...] = acc + b_ref[0]


def _tc_fuse_transform(agg, sl, wall, bias):
  return pl.pallas_call(
      _tc_fuse_transform_body,
      grid=(NB, N_RELS + 1),
      in_specs=[
          pl.BlockSpec((NC, BN, DIM), lambda i, r: (0, i, 0)),
          pl.BlockSpec((BN, DIM), lambda i, r: (i, 0)),
          pl.BlockSpec((1, DIM, DIM), lambda i, r: (r, 0, 0)),
          pl.BlockSpec((1, DIM), lambda i, r: (0, 0)),
      ],
      out_specs=[
          pl.BlockSpec((1, BN, DIM), lambda i, r: (jnp.minimum(r, N_RELS - 1), i, 0)),
          pl.BlockSpec((BN, DIM), lambda i, r: (i, 0)),
      ],
      out_shape=[
          jax.ShapeDtypeStruct((N_RELS, N_NODES, DIM), jnp.float32),
          jax.ShapeDtypeStruct((N_NODES, DIM), jnp.float32),
      ],
  )(agg, sl, wall, bias)


def _tc_final_body(a_ref, sl_ref, out_ref):
  out_ref[...] = a_ref[0] + a_ref[1] + sl_ref[...]


def _tc_final(agg, sl):
  return pl.pallas_call(
      _tc_final_body,
      grid=(NB,),
      in_specs=[
          pl.BlockSpec((NC, BN, DIM), lambda i: (0, i, 0)),
          pl.BlockSpec((BN, DIM), lambda i: (i, 0)),
      ],
      out_specs=pl.BlockSpec((BN, DIM), lambda i: (i, 0)),
      out_shape=jax.ShapeDtypeStruct((N_NODES, DIM), jnp.float32),
  )(agg, sl)


def kernel(features, edge_index, etypes, norm, W1, loop1, b1, W2, loop2, b2):
  src = edge_index[0].astype(jnp.int32)
  dst = edge_index[1].astype(jnp.int32)
  et = etypes.astype(jnp.int32)
  gidx = et * N_NODES + src

  pad = E_PAD - N_EDGES
  gidx_p = jnp.concatenate([gidx, jnp.zeros((pad,), jnp.int32)]).reshape(NW, G, GB)
  dst_p = jnp.concatenate([dst, jnp.zeros((pad,), jnp.int32)]).reshape(NW, G, GB)
  norm_p = jnp.concatenate(
      [norm.reshape(N_EDGES), jnp.zeros((pad,), jnp.float32)]).reshape(NW, G, GB)

  wall1 = jnp.concatenate([W1, loop1[None]], axis=0)
  wall2 = jnp.concatenate([W2, loop2[None]], axis=0)

  hrel1, sl1 = _tc_transform(features, wall1, b1[None])
  agg1 = _sc_gather_scatter(hrel1.reshape(N_RELS * N_NODES, DIM),
                            gidx_p, dst_p, norm_p)
  hrel2, sl2 = _tc_fuse_transform(agg1, sl1, wall2, b2[None])
  agg2 = _sc_gather_scatter(hrel2.reshape(N_RELS * N_NODES, DIM),
                            gidx_p, dst_p, norm_p)
  return _tc_final(agg2, sl2)


# speedup vs baseline: 1.0871x; 1.0267x over previous
import functools

import jax
import jax.numpy as jnp
from jax import lax
from jax.experimental import pallas as pl
from jax.experimental.pallas import tpu as pltpu
from jax.experimental.pallas import tpu_sc as plsc

N_NODES = 10000
N_EDGES = 320000
DIM = 128
N_RELS = 8

NC = 2
NS = 16
NW = NC * NS
GB = 128
G = 80
CH = 16
NCH = G // CH
EPT = G * GB
E_PAD = NW * EPT
ACC_N = 10240
ROWS_PT = ACC_N // NS

_SC_MESH = plsc.VectorSubcoreMesh(
    core_axis_name="c", subcore_axis_name="s", num_cores=NC, num_subcores=NS)


def _sc_body(hrel, gidx, dste, nrm, out, gidx_v, dst_v, norm_v, rows_v, acc,
             sg0, sg1, ss0, ss1, st0, st1):
  c = lax.axis_index("c")
  s = lax.axis_index("s")
  w = s * NC + c
  sg = (sg0, sg1)
  ss = (ss0, ss1)
  st = (st0, st1)

  def _refill(ch, buf):
    sl = pl.ds(ch * CH, CH)
    pltpu.async_copy(gidx.at[w, sl], gidx_v.at[buf], st[buf])
    pltpu.async_copy(dste.at[w, sl], dst_v.at[buf], st[buf])
    pltpu.async_copy(nrm.at[w, sl], norm_v.at[buf], st[buf])

  def _wait_refill(buf):
    for _ in range(2):
      pltpu.make_async_copy(gidx.at[w, pl.ds(0, CH)], gidx_v.at[buf],
                            st[buf]).wait()
    pltpu.make_async_copy(nrm.at[w, pl.ds(0, CH)], norm_v.at[buf],
                          st[buf]).wait()

  _refill(0, 0)
  _refill(1, 1)
  zero = jnp.zeros((16,), jnp.float32)

  def _zero_rows(e, carry):
    for j in range(DIM // 16):
      rows_v[0, e, pl.ds(j * 16, 16)] = zero
    return carry

  lax.fori_loop(0, GB, _zero_rows, 0)
  base = s * ROWS_PT
  for k in range(ROWS_PT // GB):
    pltpu.async_copy(rows_v.at[0], acc.at[pl.ds(base + k * GB, GB)], ss0)
  for k in range(ROWS_PT // GB):
    pltpu.make_async_copy(rows_v.at[0], acc.at[pl.ds(base, GB)], ss0).wait()
  _wait_refill(0)

  pltpu.async_copy(hrel.at[gidx_v.at[0, 0]], rows_v.at[0], sg0)
  plsc.subcore_barrier()

  def _scale(b, buf, gm):
    def _scale16(e16, carry2):
      nv = norm_v[buf, gm, pl.ds(e16 * 16, 16)]
      for l in range(16):
        nb = jnp.full((16,), nv[l], jnp.float32)
        e = e16 * 16 + l
        for j in range(DIM // 16):
          sl = pl.ds(j * 16, 16)
          rows_v[b, e, sl] = rows_v[b, e, sl] * nb
      return carry2

    lax.fori_loop(0, GB // 16, _scale16, 0)

  def _step(g, gm, k, buf, nbuf, gm1, last):
    bn = 1 - k
    if not (isinstance(g, int) and g == 0):
      pltpu.make_async_copy(rows_v.at[bn], acc.at[dst_v.at[0, 0]],
                            ss[bn]).wait()
    if not last:
      pltpu.async_copy(hrel.at[gidx_v.at[nbuf, gm1]], rows_v.at[bn], sg[bn])
    pltpu.make_async_copy(hrel.at[gidx_v.at[0, 0]], rows_v.at[k], sg[k]).wait()
    _scale(k, buf, gm)
    pltpu.async_copy(rows_v.at[k], acc.at[dst_v.at[buf, gm]], ss[k], add=True)

  for ch in range(NCH):
    buf = ch % 2
    g0 = ch * CH
    _step(g0, 0, 0, buf, buf, 1, False)
    _step(g0 + 1, 1, 1, buf, buf, 2, False)
    if 1 <= ch < NCH - 1:
      _refill(ch + 1, 1 - buf)

    def _mid(i, carry):
      gm = 2 + 2 * i
      _step(g0 + gm, gm, 0, buf, buf, gm + 1, False)
      _step(g0 + gm + 1, gm + 1, 1, buf, buf, gm + 2, False)
      return carry

    lax.fori_loop(0, (CH - 4) // 2, _mid, 0)
    if ch + 1 < NCH:
      _wait_refill(1 - buf)
    _step(g0 + CH - 2, CH - 2, 0, buf, buf, CH - 1, False)
    _step(g0 + CH - 1, CH - 1, 1, buf, 1 - buf, 0, ch + 1 == NCH)

  pltpu.make_async_copy(rows_v.at[1], acc.at[dst_v.at[0, 0]], ss[1]).wait()

  plsc.subcore_barrier()

  pltpu.sync_copy(acc.at[pl.ds(base, ROWS_PT)], out.at[c, pl.ds(base, ROWS_PT)])


_sc_gather_scatter = functools.partial(
    pl.kernel,
    out_type=jax.ShapeDtypeStruct((NC, ACC_N, DIM), jnp.float32),
    mesh=_SC_MESH,
    scratch_types=[
        pltpu.VMEM((2, CH, GB), jnp.int32),
        pltpu.VMEM((2, CH, GB), jnp.int32),
        pltpu.VMEM((2, CH, GB), jnp.float32),
        pltpu.VMEM((2, GB, DIM), jnp.float32),
        pltpu.VMEM_SHARED((ACC_N, DIM), jnp.float32),
        pltpu.SemaphoreType.DMA,
        pltpu.SemaphoreType.DMA,
        pltpu.SemaphoreType.DMA,
        pltpu.SemaphoreType.DMA,
        pltpu.SemaphoreType.DMA,
        pltpu.SemaphoreType.DMA,
    ],
)(_sc_body)


BN = 5000
NB = N_NODES // BN


def _tc_transform_body(x_ref, w_ref, b_ref, hrel_ref, sl_ref):
  r = pl.program_id(1)
  acc = jnp.dot(x_ref[...], w_ref[0], preferred_element_type=jnp.float32)

  @pl.when(r < N_RELS)
  def _():
    hrel_ref[0] = acc

  @pl.when(r == N_RELS)
  def _():
    sl_ref[...] = acc + b_ref[0]


def _tc_transform(x, wall, bias):
  return pl.pallas_call(
      _tc_transform_body,
      grid=(NB, N_RELS + 1),
      in_specs=[
          pl.BlockSpec((BN, DIM), lambda i, r: (i, 0)),
          pl.BlockSpec((1, DIM, DIM), lambda i, r: (r, 0, 0)),
          pl.BlockSpec((1, DIM), lambda i, r: (0, 0)),
      ],
      out_specs=[
          pl.BlockSpec((1, BN, DIM), lambda i, r: (jnp.minimum(r, N_RELS - 1), i, 0)),
          pl.BlockSpec((BN, DIM), lambda i, r: (i, 0)),
      ],
      out_shape=[
          jax.ShapeDtypeStruct((N_RELS, N_NODES, DIM), jnp.float32),
          jax.ShapeDtypeStruct((N_NODES, DIM), jnp.float32),
      ],
  )(x, wall, bias)


def _tc_fuse_transform_body(a_ref, sl_ref, w_ref, b_ref, hrel_ref, sl2_ref):
  r = pl.program_id(1)
  h = jnp.maximum(a_ref[0] + a_ref[1] + sl_ref[...], 0.0)
  acc = jnp.dot(h, w_ref[0], preferred_element_type=jnp.float32)

  @pl.when(r < N_RELS)
  def _():
    hrel_ref[0] = acc

  @pl.when(r == N_RELS)
  def _():
    sl2_ref[...] = acc + b_ref[0]


def _tc_fuse_transform(agg, sl, wall, bias):
  return pl.pallas_call(
      _tc_fuse_transform_body,
      grid=(NB, N_RELS + 1),
      in_specs=[
          pl.BlockSpec((NC, BN, DIM), lambda i, r: (0, i, 0)),
          pl.BlockSpec((BN, DIM), lambda i, r: (i, 0)),
          pl.BlockSpec((1, DIM, DIM), lambda i, r: (r, 0, 0)),
          pl.BlockSpec((1, DIM), lambda i, r: (0, 0)),
      ],
      out_specs=[
          pl.BlockSpec((1, BN, DIM), lambda i, r: (jnp.minimum(r, N_RELS - 1), i, 0)),
          pl.BlockSpec((BN, DIM), lambda i, r: (i, 0)),
      ],
      out_shape=[
          jax.ShapeDtypeStruct((N_RELS, N_NODES, DIM), jnp.float32),
          jax.ShapeDtypeStruct((N_NODES, DIM), jnp.float32),
      ],
  )(agg, sl, wall, bias)


def _tc_final_body(a_ref, sl_ref, out_ref):
  out_ref[...] = a_ref[0] + a_ref[1] + sl_ref[...]


def _tc_final(agg, sl):
  return pl.pallas_call(
      _tc_final_body,
      grid=(NB,),
      in_specs=[
          pl.BlockSpec((NC, BN, DIM), lambda i: (0, i, 0)),
          pl.BlockSpec((BN, DIM), lambda i: (i, 0)),
      ],
      out_specs=pl.BlockSpec((BN, DIM), lambda i: (i, 0)),
      out_shape=jax.ShapeDtypeStruct((N_NODES, DIM), jnp.float32),
  )(agg, sl)


def kernel(features, edge_index, etypes, norm, W1, loop1, b1, W2, loop2, b2):
  src = edge_index[0].astype(jnp.int32)
  dst = edge_index[1].astype(jnp.int32)
  et = etypes.astype(jnp.int32)
  gidx = et * N_NODES + src

  pad = E_PAD - N_EDGES
  gidx_p = jnp.concatenate([gidx, jnp.zeros((pad,), jnp.int32)]).reshape(NW, G, GB)
  dst_p = jnp.concatenate([dst, jnp.zeros((pad,), jnp.int32)]).reshape(NW, G, GB)
  norm_p = jnp.concatenate(
      [norm.reshape(N_EDGES), jnp.zeros((pad,), jnp.float32)]).reshape(NW, G, GB)

  wall1 = jnp.concatenate([W1, loop1[None]], axis=0)
  wall2 = jnp.concatenate([W2, loop2[None]], axis=0)

  hrel1, sl1 = _tc_transform(features, wall1, b1[None])
  agg1 = _sc_gather_scatter(hrel1.reshape(N_RELS * N_NODES, DIM),
                            gidx_p, dst_p, norm_p)
  hrel2, sl2 = _tc_fuse_transform(agg1, sl1, wall2, b2[None])
  agg2 = _sc_gather_scatter(hrel2.reshape(N_RELS * N_NODES, DIM),
                            gidx_p, dst_p, norm_p)
  return _tc_final(agg2, sl2)


# speedup vs baseline: 1.1017x; 1.0134x over previous
import functools

import jax
import jax.numpy as jnp
from jax import lax
from jax.experimental import pallas as pl
from jax.experimental.pallas import tpu as pltpu
from jax.experimental.pallas import tpu_sc as plsc

N_NODES = 10000
N_EDGES = 320000
DIM = 128
N_RELS = 8

NC = 2
NS = 16
NW = NC * NS
GB = 128
G = 80
CH = 16
NCH = G // CH
EPT = G * GB
E_PAD = NW * EPT
ACC_N = 10240
ROWS_PT = ACC_N // NS

_SC_MESH = plsc.VectorSubcoreMesh(
    core_axis_name="c", subcore_axis_name="s", num_cores=NC, num_subcores=NS)


def _sc_body(hrel, gidx, dste, nrm, out, gidx_v, dst_v, norm_v, rows_v, acc,
             sg0, sg1, ss0, ss1, st0, st1):
  c = lax.axis_index("c")
  s = lax.axis_index("s")
  w = s * NC + c
  sg = (sg0, sg1)
  ss = (ss0, ss1)
  st = (st0, st1)

  def _refill(ch, buf):
    sl = pl.ds(ch * CH, CH)
    pltpu.async_copy(gidx.at[w, sl], gidx_v.at[buf], st[buf])
    pltpu.async_copy(dste.at[w, sl], dst_v.at[buf], st[buf])
    pltpu.async_copy(nrm.at[w, sl], norm_v.at[buf], st[buf])

  def _wait_refill(buf):
    for _ in range(2):
      pltpu.make_async_copy(gidx.at[w, pl.ds(0, CH)], gidx_v.at[buf],
                            st[buf]).wait()
    pltpu.make_async_copy(nrm.at[w, pl.ds(0, CH)], norm_v.at[buf],
                          st[buf]).wait()

  _refill(0, 0)
  _refill(1, 1)
  zero = jnp.zeros((16,), jnp.float32)

  def _zero_rows(e, carry):
    for j in range(DIM // 16):
      rows_v[0, e, pl.ds(j * 16, 16)] = zero
    return carry

  lax.fori_loop(0, GB, _zero_rows, 0)
  base = s * ROWS_PT
  for k in range(ROWS_PT // GB):
    pltpu.async_copy(rows_v.at[0], acc.at[pl.ds(base + k * GB, GB)], ss0)
  for k in range(ROWS_PT // GB):
    pltpu.make_async_copy(rows_v.at[0], acc.at[pl.ds(base, GB)], ss0).wait()
  _wait_refill(0)

  pltpu.async_copy(hrel.at[gidx_v.at[0, 0]], rows_v.at[0], sg0)
  plsc.subcore_barrier()

  def _scale(b, buf, gm):
    def _scale16(e16, carry2):
      nv = norm_v[buf, gm, pl.ds(e16 * 16, 16)]
      for l in range(16):
        nb = jnp.full((16,), nv[l], jnp.float32)
        e = e16 * 16 + l
        for j in range(DIM // 16):
          sl = pl.ds(j * 16, 16)
          rows_v[b, e, sl] = rows_v[b, e, sl] * nb
      return carry2

    lax.fori_loop(0, GB // 16, _scale16, 0)

  def _step(g, gm, k, buf, nbuf, gm1, last):
    bn = 1 - k
    if not (isinstance(g, int) and g == 0):
      pltpu.make_async_copy(rows_v.at[bn], acc.at[dst_v.at[0, 0]],
                            ss[bn]).wait()
    if not last:
      pltpu.async_copy(hrel.at[gidx_v.at[nbuf, gm1]], rows_v.at[bn], sg[bn])
    pltpu.make_async_copy(hrel.at[gidx_v.at[0, 0]], rows_v.at[k], sg[k]).wait()
    _scale(k, buf, gm)
    pltpu.async_copy(rows_v.at[k], acc.at[dst_v.at[buf, gm]], ss[k], add=True)

  for ch in range(NCH):
    buf = ch % 2
    g0 = ch * CH
    _step(g0, 0, 0, buf, buf, 1, False)
    _step(g0 + 1, 1, 1, buf, buf, 2, False)
    if 1 <= ch < NCH - 1:
      _refill(ch + 1, 1 - buf)

    def _mid(i, carry):
      gm = 2 + 2 * i
      _step(g0 + gm, gm, 0, buf, buf, gm + 1, False)
      _step(g0 + gm + 1, gm + 1, 1, buf, buf, gm + 2, False)
      return carry

    lax.fori_loop(0, (CH - 4) // 2, _mid, 0)
    if ch + 1 < NCH:
      _wait_refill(1 - buf)
    _step(g0 + CH - 2, CH - 2, 0, buf, buf, CH - 1, False)
    _step(g0 + CH - 1, CH - 1, 1, buf, 1 - buf, 0, ch + 1 == NCH)

  pltpu.make_async_copy(rows_v.at[1], acc.at[dst_v.at[0, 0]], ss[1]).wait()

  plsc.subcore_barrier()

  pltpu.sync_copy(acc.at[pl.ds(base, ROWS_PT)], out.at[c, pl.ds(base, ROWS_PT)])


_sc_gather_scatter = functools.partial(
    pl.kernel,
    out_type=jax.ShapeDtypeStruct((NC, ACC_N, DIM), jnp.float32),
    mesh=_SC_MESH,
    scratch_types=[
        pltpu.VMEM((2, CH, GB), jnp.int32),
        pltpu.VMEM((2, CH, GB), jnp.int32),
        pltpu.VMEM((2, CH, GB), jnp.float32),
        pltpu.VMEM((2, GB, DIM), jnp.float32),
        pltpu.VMEM_SHARED((ACC_N, DIM), jnp.float32),
        pltpu.SemaphoreType.DMA,
        pltpu.SemaphoreType.DMA,
        pltpu.SemaphoreType.DMA,
        pltpu.SemaphoreType.DMA,
        pltpu.SemaphoreType.DMA,
        pltpu.SemaphoreType.DMA,
    ],
)(_sc_body)


BN = 10000
NB = N_NODES // BN


def _tc_transform_body(x_ref, w_ref, b_ref, hrel_ref, sl_ref):
  r = pl.program_id(1)
  acc = jnp.dot(x_ref[...], w_ref[0], preferred_element_type=jnp.float32)

  @pl.when(r < N_RELS)
  def _():
    hrel_ref[0] = acc

  @pl.when(r == N_RELS)
  def _():
    sl_ref[...] = acc + b_ref[0]


def _tc_transform(x, wall, bias):
  return pl.pallas_call(
      _tc_transform_body,
      grid=(NB, N_RELS + 1),
      in_specs=[
          pl.BlockSpec((BN, DIM), lambda i, r: (i, 0)),
          pl.BlockSpec((1, DIM, DIM), lambda i, r: (r, 0, 0)),
          pl.BlockSpec((1, DIM), lambda i, r: (0, 0)),
      ],
      out_specs=[
          pl.BlockSpec((1, BN, DIM), lambda i, r: (jnp.minimum(r, N_RELS - 1), i, 0)),
          pl.BlockSpec((BN, DIM), lambda i, r: (i, 0)),
      ],
      out_shape=[
          jax.ShapeDtypeStruct((N_RELS, N_NODES, DIM), jnp.float32),
          jax.ShapeDtypeStruct((N_NODES, DIM), jnp.float32),
      ],
  )(x, wall, bias)


def _tc_fuse_transform_body(a_ref, sl_ref, w_ref, b_ref, hrel_ref, sl2_ref):
  r = pl.program_id(1)
  h = jnp.maximum(a_ref[0] + a_ref[1] + sl_ref[...], 0.0)
  acc = jnp.dot(h, w_ref[0], preferred_element_type=jnp.float32)

  @pl.when(r < N_RELS)
  def _():
    hrel_ref[0] = acc

  @pl.when(r == N_RELS)
  def _():
    sl2_ref[...] = acc + b_ref[0]


def _tc_fuse_transform(agg, sl, wall, bias):
  return pl.pallas_call(
      _tc_fuse_transform_body,
      grid=(NB, N_RELS + 1),
      in_specs=[
          pl.BlockSpec((NC, BN, DIM), lambda i, r: (0, i, 0)),
          pl.BlockSpec((BN, DIM), lambda i, r: (i, 0)),
          pl.BlockSpec((1, DIM, DIM), lambda i, r: (r, 0, 0)),
          pl.BlockSpec((1, DIM), lambda i, r: (0, 0)),
      ],
      out_specs=[
          pl.BlockSpec((1, BN, DIM), lambda i, r: (jnp.minimum(r, N_RELS - 1), i, 0)),
          pl.BlockSpec((BN, DIM), lambda i, r: (i, 0)),
      ],
      out_shape=[
          jax.ShapeDtypeStruct((N_RELS, N_NODES, DIM), jnp.float32),
          jax.ShapeDtypeStruct((N_NODES, DIM), jnp.float32),
      ],
  )(agg, sl, wall, bias)


def _tc_final_body(a_ref, sl_ref, out_ref):
  out_ref[...] = a_ref[0] + a_ref[1] + sl_ref[...]


def _tc_final(agg, sl):
  return pl.pallas_call(
      _tc_final_body,
      grid=(NB,),
      in_specs=[
          pl.BlockSpec((NC, BN, DIM), lambda i: (0, i, 0)),
          pl.BlockSpec((BN, DIM), lambda i: (i, 0)),
      ],
      out_specs=pl.BlockSpec((BN, DIM), lambda i: (i, 0)),
      out_shape=jax.ShapeDtypeStruct((N_NODES, DIM), jnp.float32),
  )(agg, sl)


def kernel(features, edge_index, etypes, norm, W1, loop1, b1, W2, loop2, b2):
  src = edge_index[0].astype(jnp.int32)
  dst = edge_index[1].astype(jnp.int32)
  et = etypes.astype(jnp.int32)
  gidx = et * N_NODES + src

  pad = E_PAD - N_EDGES
  gidx_p = jnp.concatenate([gidx, jnp.zeros((pad,), jnp.int32)]).reshape(NW, G, GB)
  dst_p = jnp.concatenate([dst, jnp.zeros((pad,), jnp.int32)]).reshape(NW, G, GB)
  norm_p = jnp.concatenate(
      [norm.reshape(N_EDGES), jnp.zeros((pad,), jnp.float32)]).reshape(NW, G, GB)

  wall1 = jnp.concatenate([W1, loop1[None]], axis=0)
  wall2 = jnp.concatenate([W2, loop2[None]], axis=0)

  hrel1, sl1 = _tc_transform(features, wall1, b1[None])
  agg1 = _sc_gather_scatter(hrel1.reshape(N_RELS * N_NODES, DIM),
                            gidx_p, dst_p, norm_p)
  hrel2, sl2 = _tc_fuse_transform(agg1, sl1, wall2, b2[None])
  agg2 = _sc_gather_scatter(hrel2.reshape(N_RELS * N_NODES, DIM),
                            gidx_p, dst_p, norm_p)
  return _tc_final(agg2, sl2)
